# baseline (speedup 1.0000x reference)
import jax
import jax.numpy as jnp
from jax import lax
from jax.experimental import pallas as pl
from jax.experimental.pallas import tpu as pltpu
from jax.experimental.pallas import tpu_sc as plsc

VOCAB = 1000000
HIDDEN = 64
BATCH = 16384
SEQ = 50

HALF = VOCAB // 2

NC = 2
NS = 16
NW = NC * NS
BPW = BATCH // NW
CHUNK = BPW * SEQ

TC_BLK = 16384


def _tc_project(w_a_ref, w_c_ref, b_c_ref, t_ref, p0_ref, p1_ref):
    w2 = jnp.dot(w_a_ref[...], w_c_ref[...], preferred_element_type=jnp.float32)
    w2t = w2.T * (1.0 / SEQ)
    t = t_ref[...]
    dn = (((1,), (1,)), ((), ()))
    res_e = lax.dot_general(w2t, t[:, :HIDDEN], dn,
                            preferred_element_type=jnp.float32)
    res_o = lax.dot_general(w2t, t[:, HIDDEN:], dn,
                            preferred_element_type=jnp.float32)
    b2 = b_c_ref[...] * (1.0 / SEQ)
    p0_ref[...] = jnp.concatenate([res_e[0:1], res_o[0:1]], axis=0) + b2[0]
    p1_ref[...] = jnp.concatenate([res_e[1:2], res_o[1:2]], axis=0) + b2[1]


def _project_table(table, w_a, w_c, b_c):
    t128 = table.reshape(HALF, 2 * HIDDEN)
    grid = pl.cdiv(HALF, TC_BLK)
    p0, p1 = pl.pallas_call(
        _tc_project,
        grid=(grid,),
        in_specs=[
            pl.BlockSpec((HIDDEN, 3), lambda i: (0, 0)),
            pl.BlockSpec((3, 2), lambda i: (0, 0)),
            pl.BlockSpec((2,), lambda i: (0,)),
            pl.BlockSpec((TC_BLK, 2 * HIDDEN), lambda i: (i, 0)),
        ],
        out_specs=[
            pl.BlockSpec((2, TC_BLK), lambda i: (0, i)),
            pl.BlockSpec((2, TC_BLK), lambda i: (0, i)),
        ],
        out_shape=[
            jax.ShapeDtypeStruct((2, HALF), jnp.float32),
            jax.ShapeDtypeStruct((2, HALF), jnp.float32),
        ],
        compiler_params=pltpu.CompilerParams(
            dimension_semantics=("arbitrary",),
        ),
    )(w_a, w_c, b_c, t128)
    return p0.reshape(VOCAB), p1.reshape(VOCAB)


def _accumulate(g_ref, acc_ref):
    for t in range(BPW // 16):
        def body(l, a):
            return a + g_ref[pl.ds(l * BPW + t * 16, 16)]
        acc = lax.fori_loop(0, SEQ, body, jnp.zeros((16,), jnp.float32))
        acc_ref[pl.ds(t * 16, 16)] = acc


def _sc_body(p0_hbm, p1_hbm, ids_hbm, out0_hbm, out1_hbm,
             idx_v, g0_v, g1_v, acc_v, sem0, sem1):
    c = lax.axis_index("c")
    s = lax.axis_index("s")
    w = c * NS + s
    pltpu.sync_copy(ids_hbm.at[w], idx_v)
    cp0 = pltpu.async_copy(p0_hbm.at[idx_v], g0_v, sem0)
    cp1 = pltpu.async_copy(p1_hbm.at[idx_v], g1_v, sem1)
    cp0.wait()
    _accumulate(g0_v, acc_v)
    pltpu.sync_copy(acc_v, out0_hbm.at[pl.ds(w * BPW, BPW)])
    cp1.wait()
    _accumulate(g1_v, acc_v)
    pltpu.sync_copy(acc_v, out1_hbm.at[pl.ds(w * BPW, BPW)])


def _gather_sum(p0, p1, ids_r):
    mesh = plsc.VectorSubcoreMesh(core_axis_name="c", subcore_axis_name="s")
    fn = pl.kernel(
        _sc_body,
        out_type=[
            jax.ShapeDtypeStruct((BATCH,), jnp.float32),
            jax.ShapeDtypeStruct((BATCH,), jnp.float32),
        ],
        mesh=mesh,
        scratch_types=[
            pltpu.VMEM((CHUNK,), jnp.int32),
            pltpu.VMEM((CHUNK,), jnp.float32),
            pltpu.VMEM((CHUNK,), jnp.float32),
            pltpu.VMEM((BPW,), jnp.float32),
            pltpu.SemaphoreType.DMA,
            pltpu.SemaphoreType.DMA,
        ],
    )
    return fn(p0, p1, ids_r)


@jax.jit
def kernel(input_ids, table, W_a, W_c, b_c):
    p0, p1 = _project_table(table, W_a, W_c, b_c)
    ids = input_ids.astype(jnp.int32)
    ids = (ids & 1) * HALF + (ids >> 1)
    ids_r = (ids.reshape(NW, BPW, SEQ)
             .transpose(0, 2, 1)
             .reshape(NW, CHUNK))
    out0, out1 = _gather_sum(p0, p1, ids_r)
    return jnp.stack([out0, out1], axis=1)


# speedup vs baseline: 1.0334x; 1.0334x over previous
import jax
import jax.numpy as jnp
from jax import lax
from jax.experimental import pallas as pl
from jax.experimental.pallas import tpu as pltpu
from jax.experimental.pallas import tpu_sc as plsc

VOCAB = 1000000
HIDDEN = 64
BATCH = 16384
SEQ = 50

NC = 2
NS = 16
NW = NC * NS
BPW = BATCH // NW
CHUNK_B = 4
CHUNK_I = CHUNK_B * SEQ
NCHUNK = BPW // CHUNK_B
HREG = HIDDEN // 16


def _sum_batch(g_ref, out_ref, row0, out_row):
    def body(l, accs):
        return tuple(
            accs[k] + g_ref[row0 + l, pl.ds(k * 16, 16)] for k in range(HREG)
        )
    init = tuple(jnp.zeros((16,), jnp.float32) for _ in range(HREG))
    accs = lax.fori_loop(0, SEQ, body, init)
    for k in range(HREG):
        out_ref[out_row, pl.ds(k * 16, 16)] = accs[k]


def _sc_body(table_hbm, ids_hbm, sums_hbm, idx_v, ga_v, gb_v, out_v, sema, semb):
    c = lax.axis_index("c")
    s = lax.axis_index("s")
    w = c * NS + s
    pltpu.sync_copy(ids_hbm.at[w], idx_v)

    bufs = (ga_v, gb_v)
    sems = (sema, semb)
    pltpu.async_copy(table_hbm.at[idx_v.at[0]], ga_v, sema)
    pltpu.async_copy(table_hbm.at[idx_v.at[1]], gb_v, semb)

    def step(c2, _):
        for b in range(2):
            cc = c2 + b
            pltpu.make_async_copy(table_hbm.at[idx_v.at[cc]], bufs[b], sems[b]).wait()
            for bb in range(CHUNK_B):
                _sum_batch(bufs[b], out_v, bb * SEQ, cc * CHUNK_B + bb)
            @pl.when(cc + 2 < NCHUNK)
            def _():
                pltpu.async_copy(table_hbm.at[idx_v.at[cc + 2]], bufs[b], sems[b])
        return 0

    lax.fori_loop(0, NCHUNK // 2, lambda i, x: step(i * 2, x), 0)
    pltpu.sync_copy(out_v, sums_hbm.at[pl.ds(w * BPW, BPW)])


def _gather_sums(table, ids_r):
    mesh = plsc.VectorSubcoreMesh(core_axis_name="c", subcore_axis_name="s")
    fn = pl.kernel(
        _sc_body,
        out_type=jax.ShapeDtypeStruct((BATCH, HIDDEN), jnp.float32),
        mesh=mesh,
        scratch_types=[
            pltpu.VMEM((NCHUNK, CHUNK_I), jnp.int32),
            pltpu.VMEM((CHUNK_I, HIDDEN), jnp.float32),
            pltpu.VMEM((CHUNK_I, HIDDEN), jnp.float32),
            pltpu.VMEM((BPW, HIDDEN), jnp.float32),
            pltpu.SemaphoreType.DMA,
            pltpu.SemaphoreType.DMA,
        ],
        compiler_params=pltpu.CompilerParams(use_tc_tiling_on_sc=False),
    )
    return fn(table, ids_r)


def _tc_project(w_a_ref, w_c_ref, b_c_ref, t_ref, oe_ref, oo_ref):
    w2 = jnp.dot(w_a_ref[...], w_c_ref[...], preferred_element_type=jnp.float32)
    w2t = w2.T * (1.0 / SEQ)
    t = t_ref[...]
    dn = (((1,), (1,)), ((), ()))
    res_e = lax.dot_general(w2t, t[:, :HIDDEN], dn,
                            preferred_element_type=jnp.float32)
    res_o = lax.dot_general(w2t, t[:, HIDDEN:], dn,
                            preferred_element_type=jnp.float32)
    b2 = b_c_ref[...]
    oe_ref[...] = res_e + b2[:, None]
    oo_ref[...] = res_o + b2[:, None]


def _project_sums(sums, w_a, w_c, b_c):
    t128 = sums.reshape(BATCH // 2, 2 * HIDDEN)
    blk = 2048
    grid = (BATCH // 2) // blk
    oe, oo = pl.pallas_call(
        _tc_project,
        grid=(grid,),
        in_specs=[
            pl.BlockSpec((HIDDEN, 3), lambda i: (0, 0)),
            pl.BlockSpec((3, 2), lambda i: (0, 0)),
            pl.BlockSpec((2,), lambda i: (0,)),
            pl.BlockSpec((blk, 2 * HIDDEN), lambda i: (i, 0)),
        ],
        out_specs=[
            pl.BlockSpec((2, blk), lambda i: (0, i)),
            pl.BlockSpec((2, blk), lambda i: (0, i)),
        ],
        out_shape=[
            jax.ShapeDtypeStruct((2, BATCH // 2), jnp.float32),
            jax.ShapeDtypeStruct((2, BATCH // 2), jnp.float32),
        ],
        compiler_params=pltpu.CompilerParams(
            dimension_semantics=("arbitrary",),
        ),
    )(w_a, w_c, b_c, t128)
    return oe, oo


@jax.jit
def kernel(input_ids, table, W_a, W_c, b_c):
    ids_r = input_ids.astype(jnp.int32).reshape(NW, NCHUNK, CHUNK_I)
    sums = _gather_sums(table, ids_r)
    oe, oo = _project_sums(sums, W_a, W_c, b_c)
    le = jnp.stack([oe[0], oe[1]], axis=1)
    lo = jnp.stack([oo[0], oo[1]], axis=1)
    return jnp.stack([le, lo], axis=1).reshape(BATCH, 2)


# speedup vs baseline: 1.1665x; 1.1288x over previous
import jax
import jax.numpy as jnp
from jax import lax
from jax.experimental import pallas as pl
from jax.experimental.pallas import tpu as pltpu
from jax.experimental.pallas import tpu_sc as plsc

VOCAB = 1000000
HIDDEN = 64
BATCH = 16384
SEQ = 50

NC = 2
NS = 16
NW = NC * NS
BPW = BATCH // NW
CHUNK = BPW * SEQ

TC_BLK = 16384


def _tc_project(w_a_ref, w_c_ref, b_c_ref, t_ref, p0_ref, p1_ref):
    w2 = jnp.dot(w_a_ref[...], w_c_ref[...], preferred_element_type=jnp.float32)
    w2t = w2.T * (1.0 / SEQ)
    t = t_ref[...]
    res = lax.dot_general(w2t, t, (((1,), (1,)), ((), ())),
                          preferred_element_type=jnp.float32)
    b2 = b_c_ref[...] * (1.0 / SEQ)
    p0_ref[...] = res[0:1, :] + b2[0]
    p1_ref[...] = res[1:2, :] + b2[1]


def _project_table(table, w_a, w_c, b_c):
    grid = pl.cdiv(VOCAB, TC_BLK)
    p0, p1 = pl.pallas_call(
        _tc_project,
        grid=(grid,),
        in_specs=[
            pl.BlockSpec((HIDDEN, 3), lambda i: (0, 0)),
            pl.BlockSpec((3, 2), lambda i: (0, 0)),
            pl.BlockSpec((2,), lambda i: (0,)),
            pl.BlockSpec((TC_BLK, HIDDEN), lambda i: (i, 0)),
        ],
        out_specs=[
            pl.BlockSpec((1, TC_BLK), lambda i: (0, i)),
            pl.BlockSpec((1, TC_BLK), lambda i: (0, i)),
        ],
        out_shape=[
            jax.ShapeDtypeStruct((1, VOCAB), jnp.float32),
            jax.ShapeDtypeStruct((1, VOCAB), jnp.float32),
        ],
        compiler_params=pltpu.CompilerParams(
            dimension_semantics=("arbitrary",),
        ),
    )(w_a, w_c, b_c, table)
    return p0.reshape(VOCAB), p1.reshape(VOCAB)


def _accumulate(g_ref, acc_ref):
    for t in range(BPW // 16):
        def body(l, a):
            return a + g_ref[pl.ds(l * BPW + t * 16, 16)]
        acc = lax.fori_loop(0, SEQ, body, jnp.zeros((16,), jnp.float32))
        acc_ref[pl.ds(t * 16, 16)] = acc


def _sc_body(p0_hbm, p1_hbm, ids_hbm, out0_hbm, out1_hbm,
             idx_v, g0_v, g1_v, acc_v, sem0, sem1):
    c = lax.axis_index("c")
    s = lax.axis_index("s")
    w = c * NS + s
    pltpu.sync_copy(ids_hbm.at[w], idx_v)
    cp0 = pltpu.async_copy(p0_hbm.at[idx_v], g0_v, sem0)
    cp1 = pltpu.async_copy(p1_hbm.at[idx_v], g1_v, sem1)
    cp0.wait()
    _accumulate(g0_v, acc_v)
    pltpu.sync_copy(acc_v, out0_hbm.at[pl.ds(w * BPW, BPW)])
    cp1.wait()
    _accumulate(g1_v, acc_v)
    pltpu.sync_copy(acc_v, out1_hbm.at[pl.ds(w * BPW, BPW)])


def _gather_sum(p0, p1, ids_r):
    mesh = plsc.VectorSubcoreMesh(core_axis_name="c", subcore_axis_name="s")
    fn = pl.kernel(
        _sc_body,
        out_type=[
            jax.ShapeDtypeStruct((BATCH,), jnp.float32),
            jax.ShapeDtypeStruct((BATCH,), jnp.float32),
        ],
        mesh=mesh,
        scratch_types=[
            pltpu.VMEM((CHUNK,), jnp.int32),
            pltpu.VMEM((CHUNK,), jnp.float32),
            pltpu.VMEM((CHUNK,), jnp.float32),
            pltpu.VMEM((BPW,), jnp.float32),
            pltpu.SemaphoreType.DMA,
            pltpu.SemaphoreType.DMA,
        ],
    )
    return fn(p0, p1, ids_r)


@jax.jit
def kernel(input_ids, table, W_a, W_c, b_c):
    p0, p1 = _project_table(table, W_a, W_c, b_c)
    ids_r = (input_ids.astype(jnp.int32)
             .reshape(NW, BPW, SEQ)
             .transpose(0, 2, 1)
             .reshape(NW, CHUNK))
    out0, out1 = _gather_sum(p0, p1, ids_r)
    return jnp.stack([out0, out1], axis=1)
